# baseline (speedup 1.0000x reference)
import functools

import jax
import jax.numpy as jnp
from jax import lax
from jax.experimental import pallas as pl
from jax.experimental.pallas import tpu as pltpu
from jax.experimental.pallas import tpu_sc as plsc

HIDDEN = 64
BATCH = 16384
IDX_CHUNK = 128


@functools.cache
def _build_gather(batch: int, hidden: int):
    info = plsc.get_sparse_core_info()
    num_workers = info.num_cores * info.num_subcores
    b_per_w = batch // num_workers
    n_chunks = b_per_w // IDX_CHUNK
    mesh = plsc.VectorSubcoreMesh(core_axis_name="c", subcore_axis_name="s")

    @functools.partial(
        pl.kernel,
        mesh=mesh,
        out_type=jax.ShapeDtypeStruct((batch, hidden), jnp.float32),
        scratch_types=[
            pltpu.VMEM((b_per_w,), jnp.int32),
            pltpu.VMEM((b_per_w, hidden), jnp.float32),
            pltpu.SemaphoreType.DMA,
        ],
        compiler_params=pltpu.CompilerParams(use_tc_tiling_on_sc=False),
    )
    def gather_kernel(table_hbm, idx_hbm, out_hbm, idx_v, rows_v, sem):
        wid = lax.axis_index("s") * info.num_cores + lax.axis_index("c")
        base = wid * b_per_w
        pltpu.sync_copy(idx_hbm.at[pl.ds(base, b_per_w)], idx_v)
        copies = []
        for j in range(n_chunks):
            copies.append(
                pltpu.async_copy(
                    table_hbm.at[idx_v.at[pl.ds(j * IDX_CHUNK, IDX_CHUNK)]],
                    rows_v.at[pl.ds(j * IDX_CHUNK, IDX_CHUNK)],
                    sem,
                )
            )
        for c in copies:
            c.wait()
        pltpu.sync_copy(rows_v, out_hbm.at[pl.ds(base, b_per_w)])

    return gather_kernel


def kernel(labels, train, embedding_table):
    del train
    gather = _build_gather(BATCH, HIDDEN)
    return gather(embedding_table, labels.astype(jnp.int32))


# speedup vs baseline: 1.7176x; 1.7176x over previous
import functools

import jax
import jax.numpy as jnp
from jax import lax
from jax.experimental import pallas as pl
from jax.experimental.pallas import tpu as pltpu
from jax.experimental.pallas import tpu_sc as plsc

HIDDEN = 64
BATCH = 16384


@functools.cache
def _build_gather(batch: int, hidden: int):
    info = plsc.get_sparse_core_info()
    num_workers = info.num_cores * info.num_subcores
    b_per_w = batch // num_workers
    mesh = plsc.VectorSubcoreMesh(core_axis_name="c", subcore_axis_name="s")

    @functools.partial(
        pl.kernel,
        mesh=mesh,
        out_type=jax.ShapeDtypeStruct((batch, hidden), jnp.float32),
        scratch_types=[
            pltpu.VMEM((b_per_w,), jnp.int32),
            pltpu.VMEM((b_per_w, hidden), jnp.float32),
            pltpu.SemaphoreType.DMA,
        ],
    )
    def gather_kernel(table_hbm, idx_hbm, out_hbm, lab_v, rows_v, sem):
        wid = lax.axis_index("s") * info.num_cores + lax.axis_index("c")
        base = wid * b_per_w
        pltpu.sync_copy(idx_hbm.at[pl.ds(base, b_per_w)], lab_v)

        def body(g, carry):
            vec = lab_v[pl.ds(g * 16, 16)]
            for l in range(16):
                row = vec[l]
                pltpu.async_copy(
                    table_hbm.at[pl.ds(row, 1)],
                    rows_v.at[pl.ds(g * 16 + l, 1)],
                    sem,
                )
            return carry

        lax.fori_loop(0, b_per_w // 16, body, 0)
        pltpu.make_async_copy(
            table_hbm.at[pl.ds(0, b_per_w)], rows_v, sem
        ).wait()
        pltpu.sync_copy(rows_v, out_hbm.at[pl.ds(base, b_per_w)])

    return gather_kernel


def kernel(labels, train, embedding_table):
    del train
    gather = _build_gather(BATCH, HIDDEN)
    return gather(embedding_table, labels.astype(jnp.int32))


# speedup vs baseline: 2.4979x; 1.4543x over previous
import functools

import jax
import jax.numpy as jnp
from jax import lax
from jax.experimental import pallas as pl
from jax.experimental.pallas import tpu as pltpu
from jax.experimental.pallas import tpu_sc as plsc

HIDDEN = 64
BATCH = 16384
NUM_ROWS = 1_000_000
TAIL_START = (NUM_ROWS // 128) * 128
TC_MAX = NUM_ROWS // 128 - 1
NBUF = 4


@functools.cache
def _build_gather(batch: int, hidden: int):
    info = plsc.get_sparse_core_info()
    num_workers = info.num_cores * info.num_subcores
    b_per_w = batch // num_workers
    lab_pad = b_per_w + 32
    mesh = plsc.VectorSubcoreMesh(core_axis_name="c", subcore_axis_name="s")

    @functools.partial(
        pl.kernel,
        mesh=mesh,
        out_type=jax.ShapeDtypeStruct((hidden, batch), jnp.float32),
        scratch_types=[
            pltpu.VMEM((lab_pad,), jnp.int32),
            pltpu.VMEM((hidden, 64), jnp.float32),
            pltpu.VMEM((NBUF, hidden, 128), jnp.float32),
            pltpu.VMEM((hidden, b_per_w), jnp.float32),
            pltpu.SemaphoreType.DMA,
            pltpu.SemaphoreType.DMA,
            pltpu.SemaphoreType.DMA,
            pltpu.SemaphoreType.DMA,
        ],
        compiler_params=pltpu.CompilerParams(needs_layout_passes=False),
    )
    def gather_kernel(table_hbm, tail_hbm, idx_hbm, out_hbm, lab_v, tail_v,
                      slab_v, outb_v, *sems):
        wid = lax.axis_index("s") * info.num_cores + lax.axis_index("c")
        base = wid * b_per_w
        zeros16 = jnp.zeros((16,), jnp.int32)
        for i in range((lab_pad - b_per_w) // 16):
            lab_v[pl.ds(b_per_w + 16 * i, 16)] = zeros16
        pltpu.sync_copy(idx_hbm.at[pl.ds(base, b_per_w)],
                        lab_v.at[pl.ds(0, b_per_w)])
        pltpu.sync_copy(tail_hbm, tail_v)
        jvec = lax.iota(jnp.int32, 16)

        def issue(lab, slot):
            tc = lax.min(lax.shift_right_logical(lab, 7), TC_MAX)
            start = pl.multiple_of(tc * 128, 128)
            pltpu.async_copy(
                table_hbm.at[:, pl.ds(start, 128)], slab_v.at[slot],
                sems[slot],
            )

        def wait(slot):
            pltpu.make_async_copy(
                table_hbm.at[:, pl.ds(0, 128)], slab_v.at[slot], sems[slot]
            ).wait()

        def extract(lab, a, slot):
            avec = zeros16 + a
            is_tail = lab >= TAIL_START

            @pl.when(is_tail)
            def _():
                ctv = zeros16 + (lab - TAIL_START)
                for jc in range(hidden // 16):
                    v = plsc.load_gather(tail_v, [jvec + 16 * jc, ctv])
                    plsc.store_scatter(outb_v, [jvec + 16 * jc, avec], v)

            @pl.when(jnp.logical_not(is_tail))
            def _():
                cvec = zeros16 + (lab & 127)
                for jc in range(hidden // 16):
                    v = plsc.load_gather(
                        slab_v.at[slot], [jvec + 16 * jc, cvec]
                    )
                    plsc.store_scatter(outb_v, [jvec + 16 * jc, avec], v)

        vec0 = lab_v[pl.ds(0, 16)]
        for d in range(NBUF):
            issue(vec0[d], d)

        def group(g, carry):
            vecg = lab_v[pl.ds(g * NBUF, 16)]
            for d in range(NBUF):
                wait(d)
                extract(vecg[d], g * NBUF + d, d)
                issue(vecg[d + NBUF], d)
            return carry

        lax.fori_loop(0, b_per_w // NBUF, group, 0)
        for d in range(NBUF):
            wait(d)
        pltpu.sync_copy(outb_v, out_hbm.at[:, pl.ds(base, b_per_w)])

    return gather_kernel


def kernel(labels, train, embedding_table):
    del train
    gather = _build_gather(BATCH, HIDDEN)
    tail_t = embedding_table[TAIL_START:].T
    out_t = gather(embedding_table.T, tail_t, labels.astype(jnp.int32))
    return out_t.T


# speedup vs baseline: 2.7396x; 1.0967x over previous
import functools

import jax
import jax.numpy as jnp
from jax import lax
from jax.experimental import pallas as pl
from jax.experimental.pallas import tpu as pltpu
from jax.experimental.pallas import tpu_sc as plsc

HIDDEN = 64
BATCH = 16384
NUM_ROWS = 1_000_000
TAIL_START = (NUM_ROWS // 128) * 128
TC_MAX = NUM_ROWS // 128 - 1
NBUF = 8


@functools.cache
def _build_gather(batch: int, hidden: int):
    info = plsc.get_sparse_core_info()
    num_workers = info.num_cores * info.num_subcores
    b_per_w = batch // num_workers
    lab_pad = b_per_w + 32
    mesh = plsc.VectorSubcoreMesh(core_axis_name="c", subcore_axis_name="s")

    @functools.partial(
        pl.kernel,
        mesh=mesh,
        out_type=jax.ShapeDtypeStruct((hidden, batch), jnp.float32),
        scratch_types=[
            pltpu.VMEM((lab_pad,), jnp.int32),
            pltpu.VMEM((hidden, 64), jnp.float32),
            pltpu.VMEM((NBUF, hidden, 128), jnp.float32),
            pltpu.VMEM((hidden, b_per_w), jnp.float32),
            pltpu.SemaphoreType.DMA,
            pltpu.SemaphoreType.DMA,
            pltpu.SemaphoreType.DMA,
            pltpu.SemaphoreType.DMA,
            pltpu.SemaphoreType.DMA,
            pltpu.SemaphoreType.DMA,
            pltpu.SemaphoreType.DMA,
            pltpu.SemaphoreType.DMA,
        ],
        compiler_params=pltpu.CompilerParams(needs_layout_passes=False),
    )
    def gather_kernel(table_hbm, tail_hbm, idx_hbm, out_hbm, lab_v, tail_v,
                      slab_v, outb_v, *sems):
        wid = lax.axis_index("s") * info.num_cores + lax.axis_index("c")
        base = wid * b_per_w
        zeros16 = jnp.zeros((16,), jnp.int32)
        for i in range((lab_pad - b_per_w) // 16):
            lab_v[pl.ds(b_per_w + 16 * i, 16)] = zeros16
        pltpu.sync_copy(idx_hbm.at[pl.ds(base, b_per_w)],
                        lab_v.at[pl.ds(0, b_per_w)])
        pltpu.sync_copy(tail_hbm, tail_v)
        jvec = lax.iota(jnp.int32, 16)

        def issue(lab, slot):
            tc = lax.min(lax.shift_right_logical(lab, 7), TC_MAX)
            start = pl.multiple_of(tc * 128, 128)
            pltpu.async_copy(
                table_hbm.at[:, pl.ds(start, 128)], slab_v.at[slot],
                sems[slot],
            )

        def wait(slot):
            pltpu.make_async_copy(
                table_hbm.at[:, pl.ds(0, 128)], slab_v.at[slot], sems[slot]
            ).wait()

        def extract(lab, a, slot):
            avec = zeros16 + a
            is_tail = lab >= TAIL_START

            @pl.when(is_tail)
            def _():
                ctv = zeros16 + (lab - TAIL_START)
                for jc in range(hidden // 16):
                    v = plsc.load_gather(tail_v, [jvec + 16 * jc, ctv])
                    plsc.store_scatter(outb_v, [jvec + 16 * jc, avec], v)

            @pl.when(jnp.logical_not(is_tail))
            def _():
                cvec = zeros16 + (lab & 127)
                for jc in range(hidden // 16):
                    v = plsc.load_gather(
                        slab_v.at[slot], [jvec + 16 * jc, cvec]
                    )
                    plsc.store_scatter(outb_v, [jvec + 16 * jc, avec], v)

        vec0 = lab_v[pl.ds(0, 16)]
        for d in range(NBUF):
            issue(vec0[d], d)

        def group(g, carry):
            vecg = lab_v[pl.ds(g * NBUF, 16)]
            for d in range(NBUF):
                wait(d)
                extract(vecg[d], g * NBUF + d, d)
                issue(vecg[d + NBUF], d)
            return carry

        lax.fori_loop(0, b_per_w // NBUF, group, 0)
        for d in range(NBUF):
            wait(d)
        pltpu.sync_copy(outb_v, out_hbm.at[:, pl.ds(base, b_per_w)])

    return gather_kernel


def kernel(labels, train, embedding_table):
    del train
    gather = _build_gather(BATCH, HIDDEN)
    tail_t = embedding_table[TAIL_START:].T
    out_t = gather(embedding_table.T, tail_t, labels.astype(jnp.int32))
    return out_t.T
